# baseline (speedup 1.0000x reference)
import functools

import jax
import jax.numpy as jnp
from jax import lax
from jax.experimental import pallas as pl
from jax.experimental.pallas import tpu as pltpu
from jax.experimental.pallas import tpu_sc as plsc

BATCH = 16384
PEPTIDE = 31
VOCAB = 27
EMB = 64
ROWW = 2 * EMB

NUM_CORES = 2
NUM_SUBCORES = 16
NUM_WORKERS = NUM_CORES * NUM_SUBCORES
LANES = 16

TOKENS = BATCH * PEPTIDE
NPAIRS = TOKENS // 2
PAIRS_PER_W = NPAIRS // NUM_WORKERS
PCHUNK = 128
NPCHUNKS = PAIRS_PER_W // PCHUNK
NBUF = 4
NSLOTS = ((NPCHUNKS + NBUF - 1) // NBUF) * NBUF
EVECS = EMB // LANES

PT_ROWS = VOCAB * VOCAB * PEPTIDE
BCHUNK = 128
NBCHUNKS = (PT_ROWS + BCHUNK - 1) // BCHUNK
PT_PAD = NBCHUNKS * BCHUNK


def _make_sc_kernel():
  mesh = plsc.VectorSubcoreMesh(core_axis_name="c", subcore_axis_name="s")

  return functools.partial(
      pl.kernel,
      mesh=mesh,
      out_type=jax.ShapeDtypeStruct((NPAIRS, ROWW), jnp.float32),
      compiler_params=pltpu.CompilerParams(use_tc_tiling_on_sc=False),
      scratch_types=[
          pltpu.HBM((NUM_CORES * PT_PAD, ROWW), jnp.float32),
          pltpu.VMEM((VOCAB, EMB), jnp.float32),
          pltpu.VMEM((PEPTIDE, EMB), jnp.float32),
          pltpu.VMEM((NPCHUNKS, PCHUNK), jnp.int32),
          pltpu.VMEM((NPCHUNKS, PCHUNK), jnp.int32),
      ]
      + [pltpu.VMEM((PCHUNK, ROWW), jnp.float32) for _ in range(NBUF)]
      + [pltpu.SemaphoreType.DMA for _ in range(2 * NBUF)],
  )


def _sc_body(aa_hbm, pos_hbm, xe_hbm, xo_hbm, out_hbm,
             ptab_hbm, aav, posv, idxv, xov,
             buf0, buf1, buf2, buf3,
             gsem0, gsem1, gsem2, gsem3,
             wsem0, wsem1, wsem2, wsem3):
  bufs = (buf0, buf1, buf2, buf3)
  gsems = (gsem0, gsem1, gsem2, gsem3)
  wsems = (wsem0, wsem1, wsem2, wsem3)

  cid = lax.axis_index("c")
  sid = lax.axis_index("s")
  wid = sid * NUM_CORES + cid
  pair0 = wid * PAIRS_PER_W
  scbase = cid * PT_PAD

  pltpu.sync_copy(xe_hbm.at[wid], idxv)
  pltpu.sync_copy(xo_hbm.at[wid], xov)
  pltpu.sync_copy(aa_hbm, aav)
  pltpu.sync_copy(pos_hbm, posv)

  def build_row(r01, c, carry):
    rr = jnp.minimum(c * BCHUNK + r01, PT_ROWS - 1)
    v1 = lax.div(rr, VOCAB * PEPTIDE)
    re = lax.rem(rr, VOCAB * PEPTIDE)
    v2 = lax.div(re, PEPTIDE)
    p1 = lax.rem(re, PEPTIDE)
    p2 = lax.rem(p1 + 1, PEPTIDE)
    for e in range(EVECS):
      col = e * LANES
      buf0[r01, pl.ds(col, LANES)] = (
          aav[v1, pl.ds(col, LANES)] + posv[p1, pl.ds(col, LANES)])
      buf0[r01, pl.ds(EMB + col, LANES)] = (
          aav[v2, pl.ds(col, LANES)] + posv[p2, pl.ds(col, LANES)])
    return carry

  for ci in range(NBCHUNKS // NUM_SUBCORES + 1):
    c = sid + ci * NUM_SUBCORES

    @pl.when(c < NBCHUNKS)
    def _():
      lax.fori_loop(0, BCHUNK, lambda r01, cr: build_row(r01, c, cr), 0)
      pltpu.sync_copy(buf0,
                      ptab_hbm.at[pl.ds(scbase + c * BCHUNK, BCHUNK)])

  plsc.subcore_barrier()

  iota = lax.iota(jnp.int32, LANES)

  def build_idx(c, carry):
    for k in range(PCHUNK // LANES):
      col = k * LANES
      p1 = lax.rem(2 * (c * PCHUNK + col + iota), PEPTIDE)
      idxv[c, pl.ds(col, LANES)] = (
          (idxv[c, pl.ds(col, LANES)] * VOCAB + xov[c, pl.ds(col, LANES)])
          * PEPTIDE + p1 + scbase)
    return carry

  lax.fori_loop(0, NPCHUNKS, build_idx, 0)

  def start_gather(c, b):
    pltpu.async_copy(ptab_hbm.at[idxv.at[c]], bufs[b], gsems[b])

  def wait_gather(c, b):
    pltpu.make_async_copy(ptab_hbm.at[idxv.at[c]], bufs[b], gsems[b]).wait()

  def start_write(c, b):
    pltpu.async_copy(bufs[b], out_hbm.at[pl.ds(pair0 + c * PCHUNK, PCHUNK)],
                     wsems[b])

  def drain_write(b):
    pltpu.make_async_copy(bufs[b], out_hbm.at[pl.ds(pair0, PCHUNK)],
                          wsems[b]).wait()

  start_gather(0, 0)
  start_gather(1, 1)

  def chunk_step(i, carry):
    for b in range(NBUF):
      c = i * NBUF + b
      bn = (b + 2) % NBUF

      @pl.when(jnp.logical_and(c >= 2, c + 2 < NPCHUNKS))
      def _():
        drain_write(bn)

      @pl.when(c + 2 < NPCHUNKS)
      def _():
        start_gather(c + 2, bn)

      @pl.when(c < NPCHUNKS)
      def _():
        wait_gather(c, b)
        start_write(c, b)
    return carry

  lax.fori_loop(0, NSLOTS // NBUF, chunk_step, 0)

  for b in range(NBUF):
    drain_write(b)


def kernel(x, aa_table, pos_table):
  xf = x.astype(jnp.int32).reshape(TOKENS)
  xe = xf[0::2].reshape(NUM_WORKERS, NPCHUNKS, PCHUNK)
  xo = xf[1::2].reshape(NUM_WORKERS, NPCHUNKS, PCHUNK)
  out = _make_sc_kernel()(_sc_body)(aa_table, pos_table, xe, xo)
  return out.reshape(BATCH, PEPTIDE, EMB)


# speedup vs baseline: 1.2949x; 1.2949x over previous
import functools

import jax
import jax.numpy as jnp
from jax import lax
from jax.experimental import pallas as pl
from jax.experimental.pallas import tpu as pltpu
from jax.experimental.pallas import tpu_sc as plsc

BATCH = 16384
PEPTIDE = 31
VOCAB = 27
EMB = 64

NUM_CORES = 2
NUM_SUBCORES = 16
NUM_WORKERS = NUM_CORES * NUM_SUBCORES
LANES = 16

TROWS = VOCAB * PEPTIDE
TOKENS = BATCH * PEPTIDE
TOK_PER_W = TOKENS // NUM_WORKERS
ROWS_PER_W = BATCH // NUM_WORKERS
GROUP = 4
NGROUPS = ROWS_PER_W // GROUP
NBUF = 4
IDXW = PEPTIDE
EVECS = EMB // LANES


def _make_sc_kernel():
  mesh = plsc.VectorSubcoreMesh(core_axis_name="c", subcore_axis_name="s")

  return functools.partial(
      pl.kernel,
      mesh=mesh,
      out_type=jax.ShapeDtypeStruct((BATCH, PEPTIDE, EMB), jnp.float32),
      compiler_params=pltpu.CompilerParams(use_tc_tiling_on_sc=False),
      scratch_types=[
          pltpu.HBM((NUM_WORKERS * TROWS, EMB), jnp.float32),
          pltpu.VMEM((TROWS, EMB), jnp.float32),
          pltpu.VMEM((VOCAB, EMB), jnp.float32),
          pltpu.VMEM((PEPTIDE, EMB), jnp.float32),
          pltpu.VMEM((TOK_PER_W + LANES,), jnp.int32),
          pltpu.VMEM((ROWS_PER_W, IDXW), jnp.int32),
      ]
      + [pltpu.VMEM((GROUP, PEPTIDE, EMB), jnp.float32) for _ in range(NBUF)]
      + [pltpu.SemaphoreType.DMA for _ in range(2 * NBUF)],
  )


def _sc_body(aa_hbm, pos_hbm, x_hbm, out_hbm,
             ctab_hbm, tabv, aav, posv, xv, idxv,
             buf0, buf1, buf2, buf3,
             gsem0, gsem1, gsem2, gsem3,
             wsem0, wsem1, wsem2, wsem3):
  bufs = (buf0, buf1, buf2, buf3)
  gsems = (gsem0, gsem1, gsem2, gsem3)
  wsems = (wsem0, wsem1, wsem2, wsem3)

  wid = lax.axis_index("s") * NUM_CORES + lax.axis_index("c")
  row0 = wid * ROWS_PER_W
  tbase = wid * TROWS

  pltpu.sync_copy(x_hbm.at[wid], xv.at[pl.ds(0, TOK_PER_W)])
  pltpu.sync_copy(aa_hbm, aav)
  pltpu.sync_copy(pos_hbm, posv)

  def build_vocab(v, carry):
    avecs = [aav[v, pl.ds(e * LANES, LANES)] for e in range(EVECS)]

    def build_pos(p, carry2):
      for e in range(EVECS):
        tabv[v * PEPTIDE + p, pl.ds(e * LANES, LANES)] = (
            avecs[e] + posv[p, pl.ds(e * LANES, LANES)])
      return carry2

    lax.fori_loop(0, PEPTIDE, build_pos, 0)
    return carry

  lax.fori_loop(0, VOCAB, build_vocab, 0)

  pltpu.sync_copy(tabv, ctab_hbm.at[pl.ds(tbase, TROWS)])

  iota = lax.iota(jnp.int32, LANES)
  base_lo = iota + tbase
  base_hi = iota + (15 + tbase)

  def build_idx(g, carry):
    x_lo = xv[pl.ds(g * PEPTIDE, LANES)]
    x_hi = xv[pl.ds(g * PEPTIDE + 15, LANES)]
    idxv[g, pl.ds(0, LANES)] = x_lo * PEPTIDE + base_lo
    idxv[g, pl.ds(15, LANES)] = x_hi * PEPTIDE + base_hi
    return carry

  lax.fori_loop(0, ROWS_PER_W, build_idx, 0)

  def start_gathers(gg, b):
    for bb in range(GROUP):
      pltpu.async_copy(
          ctab_hbm.at[idxv.at[gg * GROUP + bb]],
          bufs[b].at[bb], gsems[b])

  def wait_gathers(gg, b):
    for bb in range(GROUP):
      pltpu.make_async_copy(
          ctab_hbm.at[idxv.at[gg * GROUP + bb]],
          bufs[b].at[bb], gsems[b]).wait()

  def start_write(gg, b):
    pltpu.async_copy(bufs[b], out_hbm.at[pl.ds(row0 + gg * GROUP, GROUP)],
                     wsems[b])

  def drain_write(b):
    pltpu.make_async_copy(bufs[b], out_hbm.at[pl.ds(row0, GROUP)],
                          wsems[b]).wait()

  start_gathers(0, 0)
  start_gathers(1, 1)

  def group_step(i, carry):
    for b in range(NBUF):
      gg = i * NBUF + b
      bn = (b + 2) % NBUF

      @pl.when(jnp.logical_and(gg >= 2, gg + 2 < NGROUPS))
      def _():
        drain_write(bn)

      @pl.when(gg + 2 < NGROUPS)
      def _():
        start_gathers(gg + 2, bn)

      wait_gathers(gg, b)
      start_write(gg, b)
    return carry

  lax.fori_loop(0, NGROUPS // NBUF, group_step, 0)

  for b in range(NBUF):
    drain_write(b)


def kernel(x, aa_table, pos_table):
  x2 = x.astype(jnp.int32).reshape(NUM_WORKERS, TOK_PER_W)
  return _make_sc_kernel()(_sc_body)(aa_table, pos_table, x2)


# speedup vs baseline: 1.5389x; 1.1884x over previous
import functools

import jax
import jax.numpy as jnp
from jax import lax
from jax.experimental import pallas as pl
from jax.experimental.pallas import tpu as pltpu
from jax.experimental.pallas import tpu_sc as plsc

BATCH = 16384
PEPTIDE = 31
VOCAB = 27
EMB = 64
PPAD = PEPTIDE + 1
ROWW = 2 * EMB

NUM_CORES = 2
NUM_SUBCORES = 16
NUM_WORKERS = NUM_CORES * NUM_SUBCORES
LANES = 16

TROWS = VOCAB * PEPTIDE
BCHUNK = 128
NBCHUNKS = (TROWS + BCHUNK - 1) // BCHUNK
TPAD = NBCHUNKS * BCHUNK
TOKENS = BATCH * PEPTIDE
TOK_PER_W = TOKENS // NUM_WORKERS
ROWS_PER_W = BATCH // NUM_WORKERS
GROUP = 4
NGROUPS = ROWS_PER_W // GROUP
NBUF = 4
EVECS = EMB // LANES


def _make_sc_kernel():
  mesh = plsc.VectorSubcoreMesh(core_axis_name="c", subcore_axis_name="s")

  return functools.partial(
      pl.kernel,
      mesh=mesh,
      out_type=jax.ShapeDtypeStruct((BATCH, PPAD, ROWW), jnp.float32),
      compiler_params=pltpu.CompilerParams(use_tc_tiling_on_sc=False),
      scratch_types=[
          pltpu.HBM((NUM_WORKERS * TPAD, ROWW), jnp.float32),
          pltpu.VMEM((BCHUNK, ROWW), jnp.float32),
          pltpu.VMEM((VOCAB, EMB), jnp.float32),
          pltpu.VMEM((PEPTIDE, EMB), jnp.float32),
          pltpu.VMEM((TOK_PER_W + LANES,), jnp.int32),
          pltpu.VMEM((ROWS_PER_W, PPAD), jnp.int32),
      ]
      + [pltpu.VMEM((GROUP, PPAD, ROWW), jnp.float32) for _ in range(NBUF)]
      + [pltpu.SemaphoreType.DMA for _ in range(2 * NBUF)],
  )


def _sc_body(aa_hbm, pos_hbm, x_hbm, out_hbm,
             ctab_hbm, bldv, aav, posv, xv, idxv,
             buf0, buf1, buf2, buf3,
             gsem0, gsem1, gsem2, gsem3,
             wsem0, wsem1, wsem2, wsem3):
  bufs = (buf0, buf1, buf2, buf3)
  gsems = (gsem0, gsem1, gsem2, gsem3)
  wsems = (wsem0, wsem1, wsem2, wsem3)

  wid = lax.axis_index("s") * NUM_CORES + lax.axis_index("c")
  row0 = wid * ROWS_PER_W
  tbase = wid * TPAD

  pltpu.sync_copy(x_hbm.at[wid], xv.at[pl.ds(0, TOK_PER_W)])
  pltpu.sync_copy(aa_hbm, aav)
  pltpu.sync_copy(pos_hbm, posv)

  def build_row(r01, c, carry):
    rr = jnp.minimum(c * BCHUNK + r01, TROWS - 1)
    v = lax.div(rr, PEPTIDE)
    p = lax.rem(rr, PEPTIDE)
    for e in range(EVECS):
      col = e * LANES
      bldv[r01, pl.ds(col, LANES)] = (
          aav[v, pl.ds(col, LANES)] + posv[p, pl.ds(col, LANES)])
    return carry

  for c in range(NBCHUNKS):
    lax.fori_loop(0, BCHUNK, lambda r01, cr: build_row(r01, c, cr), 0)
    pltpu.sync_copy(bldv, ctab_hbm.at[pl.ds(tbase + c * BCHUNK, BCHUNK)])

  iota = lax.iota(jnp.int32, LANES)
  base_lo = iota + tbase
  base_hi = iota + (LANES + tbase)

  def build_idx(g, carry):
    x_lo = xv[pl.ds(g * PEPTIDE, LANES)]
    x_hi = xv[pl.ds(g * PEPTIDE + LANES, LANES)]
    idxv[g, pl.ds(0, LANES)] = x_lo * PEPTIDE + base_lo
    idxv[g, pl.ds(LANES, LANES)] = jnp.where(
        iota < LANES - 1, x_hi * PEPTIDE + base_hi, tbase)
    return carry

  lax.fori_loop(0, ROWS_PER_W, build_idx, 0)

  def start_gathers(gg, b):
    for bb in range(GROUP):
      pltpu.async_copy(ctab_hbm.at[idxv.at[gg * GROUP + bb]],
                       bufs[b].at[bb], gsems[b])

  def wait_gathers(gg, b):
    for bb in range(GROUP):
      pltpu.make_async_copy(ctab_hbm.at[idxv.at[gg * GROUP + bb]],
                            bufs[b].at[bb], gsems[b]).wait()

  def start_write(gg, b):
    pltpu.async_copy(bufs[b], out_hbm.at[pl.ds(row0 + gg * GROUP, GROUP)],
                     wsems[b])

  def drain_write(b):
    pltpu.make_async_copy(bufs[b], out_hbm.at[pl.ds(row0, GROUP)],
                          wsems[b]).wait()

  start_gathers(0, 0)
  start_gathers(1, 1)

  def group_step(i, carry):
    for b in range(NBUF):
      gg = i * NBUF + b
      bn = (b + 2) % NBUF

      @pl.when(jnp.logical_and(gg >= 2, gg + 2 < NGROUPS))
      def _():
        drain_write(bn)

      @pl.when(gg + 2 < NGROUPS)
      def _():
        start_gathers(gg + 2, bn)

      wait_gathers(gg, b)
      start_write(gg, b)
    return carry

  lax.fori_loop(0, NGROUPS // NBUF, group_step, 0)

  for b in range(NBUF):
    drain_write(b)


def kernel(x, aa_table, pos_table):
  x2 = x.astype(jnp.int32).reshape(NUM_WORKERS, TOK_PER_W)
  out = _make_sc_kernel()(_sc_body)(aa_table, pos_table, x2)
  return out[:, :PEPTIDE, :EMB]
